# speedup vs baseline: 6.6535x; 6.6535x over previous
import functools

import jax
import jax.numpy as jnp
from jax import lax
from jax.experimental import pallas as pl
from jax.experimental.pallas import tpu as pltpu
from jax.experimental.pallas import tpu_sc as plsc

_MAXT = 1.0
_ROWS = 50000
_DIM = 128
_DELTAT = _MAXT / _ROWS
_LANES = 16
_SUB = 128


@functools.cache
def _sc_gather(R, C, NC, NS):
    NW = NC * NS
    b_per_w = R // NW
    n_chunks = b_per_w // C
    n_sub = C // _SUB
    mesh = plsc.VectorSubcoreMesh(core_axis_name="c", subcore_axis_name="s")

    @functools.partial(
        pl.kernel,
        out_type=jax.ShapeDtypeStruct((R, _DIM), jnp.float32),
        mesh=mesh,
        scratch_types=[
            pltpu.VMEM((C,), jnp.float32),
            pltpu.VMEM((C,), jnp.int32),
            pltpu.VMEM((C, _DIM), jnp.float32),
            pltpu.SemaphoreType.DMA,
        ],
    )
    def k(ts_hbm, table_hbm, out_hbm, ts_v, idx_v, rows_v, sem):
        wid = lax.axis_index("s") * NC + lax.axis_index("c")
        base = wid * b_per_w

        def chunk_body(g, carry):
            start = base + g * C
            pltpu.sync_copy(ts_hbm.at[pl.ds(start, C)], ts_v)

            def idx_body(i, c):
                v = ts_v[pl.ds(i * _LANES, _LANES)]
                q = (v / _DELTAT).astype(jnp.int32)
                q = jnp.minimum(jnp.maximum(q, 0), _ROWS - 1)
                idx_v[pl.ds(i * _LANES, _LANES)] = q
                return c

            lax.fori_loop(0, C // _LANES, idx_body, 0)

            cps = [
                pltpu.async_copy(
                    table_hbm.at[idx_v.at[pl.ds(j * _SUB, _SUB)]],
                    rows_v.at[pl.ds(j * _SUB, _SUB)],
                    sem,
                )
                for j in range(n_sub)
            ]
            for cp in cps:
                cp.wait()
            pltpu.sync_copy(rows_v, out_hbm.at[pl.ds(start, C)])
            return carry

        lax.fori_loop(0, n_chunks, chunk_body, 0)

    return k


def kernel(timestamps, table):
    B, T = timestamps.shape
    R = B * T
    info = plsc.get_sparse_core_info()
    k = _sc_gather(R, 256, info.num_cores, info.num_subcores)
    out = k(jnp.reshape(timestamps, (R,)), table)
    return jnp.reshape(out, (B, T, _DIM))


# speedup vs baseline: 9.2287x; 1.3870x over previous
import functools

import jax
import jax.numpy as jnp
from jax import lax
from jax.experimental import pallas as pl
from jax.experimental.pallas import tpu as pltpu
from jax.experimental.pallas import tpu_sc as plsc

_MAXT = 1.0
_ROWS = 50000
_DIM = 128
_DELTAT = _MAXT / _ROWS
_LANES = 16
_SUB = 128
_NBUF = 2


@functools.cache
def _sc_gather(R, C, NC, NS):
    NW = NC * NS
    b_per_w = R // NW
    n_chunks = b_per_w // C
    n_sub = C // _SUB
    assert n_chunks % _NBUF == 0
    mesh = plsc.VectorSubcoreMesh(core_axis_name="c", subcore_axis_name="s")

    buf_types = []
    for _ in range(_NBUF):
        buf_types += [
            pltpu.VMEM((C,), jnp.float32),
            pltpu.VMEM((C,), jnp.int32),
            pltpu.VMEM((C, _DIM), jnp.float32),
            pltpu.SemaphoreType.DMA,
            pltpu.SemaphoreType.DMA,
        ]

    @functools.partial(
        pl.kernel,
        out_type=jax.ShapeDtypeStruct((R, _DIM), jnp.float32),
        mesh=mesh,
        scratch_types=buf_types,
    )
    def k(ts_hbm, table_hbm, out_hbm, *bufs):
        wid = lax.axis_index("s") * NC + lax.axis_index("c")
        base = wid * b_per_w
        ts_v = [bufs[5 * b + 0] for b in range(_NBUF)]
        idx_v = [bufs[5 * b + 1] for b in range(_NBUF)]
        rows_v = [bufs[5 * b + 2] for b in range(_NBUF)]
        gsem = [bufs[5 * b + 3] for b in range(_NBUF)]
        osem = [bufs[5 * b + 4] for b in range(_NBUF)]

        def stage(chunk, b):
            start = base + chunk * C
            pltpu.sync_copy(ts_hbm.at[pl.ds(start, C)], ts_v[b])

            def idx_body(i, c):
                v = ts_v[b][pl.ds(i * _LANES, _LANES)]
                q = (v / _DELTAT).astype(jnp.int32)
                q = jnp.minimum(jnp.maximum(q, 0), _ROWS - 1)
                idx_v[b][pl.ds(i * _LANES, _LANES)] = q
                return c

            lax.fori_loop(0, C // _LANES, idx_body, 0)
            for j in range(n_sub):
                pltpu.async_copy(
                    table_hbm.at[idx_v[b].at[pl.ds(j * _SUB, _SUB)]],
                    rows_v[b].at[pl.ds(j * _SUB, _SUB)],
                    gsem[b],
                )

        def wait_gather(b):
            for j in range(n_sub):
                pltpu.make_async_copy(
                    table_hbm.at[idx_v[b].at[pl.ds(j * _SUB, _SUB)]],
                    rows_v[b].at[pl.ds(j * _SUB, _SUB)],
                    gsem[b],
                ).wait()

        def fire_scatter(chunk, b):
            start = base + chunk * C
            pltpu.async_copy(rows_v[b], out_hbm.at[pl.ds(start, C)], osem[b])

        def wait_scatter(chunk, b):
            start = base + chunk * C
            pltpu.make_async_copy(
                rows_v[b], out_hbm.at[pl.ds(start, C)], osem[b]
            ).wait()

        for b in range(_NBUF):
            stage(b, b)

        def pair_body(g, carry):
            for b in range(_NBUF):
                chunk = g * _NBUF + b
                wait_gather(b)
                fire_scatter(chunk, b)
                nxt = chunk + _NBUF
                start2 = base + nxt * C
                pltpu.sync_copy(ts_hbm.at[pl.ds(start2, C)], ts_v[b])

                def idx_body(i, c, b=b):
                    v = ts_v[b][pl.ds(i * _LANES, _LANES)]
                    q = (v / _DELTAT).astype(jnp.int32)
                    q = jnp.minimum(jnp.maximum(q, 0), _ROWS - 1)
                    idx_v[b][pl.ds(i * _LANES, _LANES)] = q
                    return c

                lax.fori_loop(0, C // _LANES, idx_body, 0)
                wait_scatter(chunk, b)
                for j in range(n_sub):
                    pltpu.async_copy(
                        table_hbm.at[idx_v[b].at[pl.ds(j * _SUB, _SUB)]],
                        rows_v[b].at[pl.ds(j * _SUB, _SUB)],
                        gsem[b],
                    )
            return carry

        lax.fori_loop(0, n_chunks // _NBUF - 1, pair_body, 0)

        for b in range(_NBUF):
            chunk = n_chunks - _NBUF + b
            wait_gather(b)
            fire_scatter(chunk, b)
        for b in range(_NBUF):
            wait_scatter(n_chunks - _NBUF + b, b)

    return k


def kernel(timestamps, table):
    B, T = timestamps.shape
    R = B * T
    info = plsc.get_sparse_core_info()
    k = _sc_gather(R, 256, info.num_cores, info.num_subcores)
    out = k(jnp.reshape(timestamps, (R,)), table)
    return jnp.reshape(out, (B, T, _DIM))


# speedup vs baseline: 9.2879x; 1.0064x over previous
import functools

import jax
import jax.numpy as jnp
from jax import lax
from jax.experimental import pallas as pl
from jax.experimental.pallas import tpu as pltpu
from jax.experimental.pallas import tpu_sc as plsc

_MAXT = 1.0
_ROWS = 50000
_DIM = 128
_DELTAT = _MAXT / _ROWS
_LANES = 16
_SUB = 128
_NBUF = 4


@functools.cache
def _sc_gather(R, C, NC, NS):
    NW = NC * NS
    b_per_w = R // NW
    n_chunks = b_per_w // C
    n_sub = C // _SUB
    assert n_chunks % _NBUF == 0
    mesh = plsc.VectorSubcoreMesh(core_axis_name="c", subcore_axis_name="s")

    buf_types = []
    for _ in range(_NBUF):
        buf_types += [
            pltpu.VMEM((C,), jnp.float32),
            pltpu.VMEM((C,), jnp.int32),
            pltpu.VMEM((C, _DIM), jnp.float32),
            pltpu.SemaphoreType.DMA,
            pltpu.SemaphoreType.DMA,
        ]

    @functools.partial(
        pl.kernel,
        out_type=jax.ShapeDtypeStruct((R, _DIM), jnp.float32),
        mesh=mesh,
        scratch_types=buf_types,
    )
    def k(ts_hbm, table_hbm, out_hbm, *bufs):
        wid = lax.axis_index("s") * NC + lax.axis_index("c")
        base = wid * b_per_w
        ts_v = [bufs[5 * b + 0] for b in range(_NBUF)]
        idx_v = [bufs[5 * b + 1] for b in range(_NBUF)]
        rows_v = [bufs[5 * b + 2] for b in range(_NBUF)]
        gsem = [bufs[5 * b + 3] for b in range(_NBUF)]
        osem = [bufs[5 * b + 4] for b in range(_NBUF)]

        def stage(chunk, b):
            start = base + chunk * C
            pltpu.sync_copy(ts_hbm.at[pl.ds(start, C)], ts_v[b])

            def idx_body(i, c):
                v = ts_v[b][pl.ds(i * _LANES, _LANES)]
                q = (v / _DELTAT).astype(jnp.int32)
                q = jnp.minimum(jnp.maximum(q, 0), _ROWS - 1)
                idx_v[b][pl.ds(i * _LANES, _LANES)] = q
                return c

            lax.fori_loop(0, C // _LANES, idx_body, 0)
            for j in range(n_sub):
                pltpu.async_copy(
                    table_hbm.at[idx_v[b].at[pl.ds(j * _SUB, _SUB)]],
                    rows_v[b].at[pl.ds(j * _SUB, _SUB)],
                    gsem[b],
                )

        def wait_gather(b):
            for j in range(n_sub):
                pltpu.make_async_copy(
                    table_hbm.at[idx_v[b].at[pl.ds(j * _SUB, _SUB)]],
                    rows_v[b].at[pl.ds(j * _SUB, _SUB)],
                    gsem[b],
                ).wait()

        def fire_scatter(chunk, b):
            start = base + chunk * C
            pltpu.async_copy(rows_v[b], out_hbm.at[pl.ds(start, C)], osem[b])

        def wait_scatter(chunk, b):
            start = base + chunk * C
            pltpu.make_async_copy(
                rows_v[b], out_hbm.at[pl.ds(start, C)], osem[b]
            ).wait()

        for b in range(_NBUF):
            stage(b, b)

        def pair_body(g, carry):
            for b in range(_NBUF):
                chunk = g * _NBUF + b
                wait_gather(b)
                fire_scatter(chunk, b)
                nxt = chunk + _NBUF
                start2 = base + nxt * C
                pltpu.sync_copy(ts_hbm.at[pl.ds(start2, C)], ts_v[b])

                def idx_body(i, c, b=b):
                    v = ts_v[b][pl.ds(i * _LANES, _LANES)]
                    q = (v / _DELTAT).astype(jnp.int32)
                    q = jnp.minimum(jnp.maximum(q, 0), _ROWS - 1)
                    idx_v[b][pl.ds(i * _LANES, _LANES)] = q
                    return c

                lax.fori_loop(0, C // _LANES, idx_body, 0)
                wait_scatter(chunk, b)
                for j in range(n_sub):
                    pltpu.async_copy(
                        table_hbm.at[idx_v[b].at[pl.ds(j * _SUB, _SUB)]],
                        rows_v[b].at[pl.ds(j * _SUB, _SUB)],
                        gsem[b],
                    )
            return carry

        lax.fori_loop(0, n_chunks // _NBUF - 1, pair_body, 0)

        for b in range(_NBUF):
            chunk = n_chunks - _NBUF + b
            wait_gather(b)
            fire_scatter(chunk, b)
        for b in range(_NBUF):
            wait_scatter(n_chunks - _NBUF + b, b)

    return k


def kernel(timestamps, table):
    B, T = timestamps.shape
    R = B * T
    info = plsc.get_sparse_core_info()
    k = _sc_gather(R, 128, info.num_cores, info.num_subcores)
    out = k(jnp.reshape(timestamps, (R,)), table)
    return jnp.reshape(out, (B, T, _DIM))
